# baseline (speedup 1.0000x reference)
import functools
import jax
import jax.numpy as jnp
from jax import lax
from jax.experimental import pallas as pl
from jax.experimental.pallas import tpu as pltpu
from jax.experimental.pallas import tpu_sc as plsc

F32 = jnp.float32
BF16 = jnp.bfloat16


def _prep_body(x_ref, lng_ref, lnb_ref, ltl_ref, wa_ref, wb_ref, wl_ref,
               be1_ref, h_ref, a_ref, b_ref, latc_ref):
    x = x_ref[...]
    mu = jnp.mean(x, axis=-1, keepdims=True)
    var = jnp.mean((x - mu) ** 2, axis=-1, keepdims=True)
    h = (x - mu) * lax.rsqrt(var + 1e-5) * lng_ref[...] + lnb_ref[...]
    h_ref[...] = h
    a_ref[...] = jnp.dot(h, wa_ref[...], preferred_element_type=F32)
    b_ref[...] = jnp.dot(h, wb_ref[...], preferred_element_type=F32)
    latc_ref[...] = jnp.dot(ltl_ref[...], wl_ref[...],
                            preferred_element_type=F32) + be1_ref[...]


def _tc_prep(x, lng, lnb, ltl, wa, wb, wl, be1):
    n, d = x.shape
    g = ltl.shape[0]
    return pl.pallas_call(
        _prep_body,
        out_shape=(
            jax.ShapeDtypeStruct((n, d), F32),
            jax.ShapeDtypeStruct((n, d), F32),
            jax.ShapeDtypeStruct((n, d), F32),
            jax.ShapeDtypeStruct((g, d), F32),
        ),
    )(x, lng, lnb, ltl, wa, wb, wl, be1)


def _edge_body(pre_ref, fr_ref, wf_ref, w2_ref, be2_ref, out_ref):
    x = pre_ref[...] + jnp.dot(fr_ref[...], wf_ref[...],
                               preferred_element_type=F32)
    e1 = x * jax.nn.sigmoid(x)
    y = jnp.dot(e1, w2_ref[...], preferred_element_type=F32) + be2_ref[...]
    out_ref[...] = y * jax.nn.sigmoid(y)


def _tc_edge(pre, fracp, wf, w2, be2, blk=2048):
    e, d = pre.shape
    grid = e // blk
    return pl.pallas_call(
        _edge_body,
        grid=(grid,),
        in_specs=[
            pl.BlockSpec((blk, d), lambda i: (i, 0)),
            pl.BlockSpec((blk, 8), lambda i: (i, 0)),
            pl.BlockSpec((8, d), lambda i: (0, 0)),
            pl.BlockSpec((d, d), lambda i: (0, 0)),
            pl.BlockSpec((1, d), lambda i: (0, 0)),
        ],
        out_specs=pl.BlockSpec((blk, d), lambda i: (i, 0)),
        out_shape=jax.ShapeDtypeStruct((e, d), F32),
    )(pre, fracp, wf, w2, be2)


def _node_body(x_ref, h_ref, aggp_ref, cntp_ref, wh_ref, wg_ref, bn1_ref,
               wn2_ref, bn2_ref, out_ref):
    n = x_ref.shape[0]
    agg = aggp_ref[0, 0:n, :] + aggp_ref[1, 0:n, :]
    cnt = cntp_ref[0, 0:n, 0:1] + cntp_ref[1, 0:n, 0:1]
    agg = agg / jnp.maximum(cnt, 1.0)
    h = h_ref[...]
    t = (jnp.dot(h, wh_ref[...], preferred_element_type=F32)
         + jnp.dot(agg, wg_ref[...], preferred_element_type=F32)
         + bn1_ref[...])
    t = t * jax.nn.sigmoid(t)
    y = jnp.dot(t, wn2_ref[...], preferred_element_type=F32) + bn2_ref[...]
    out_ref[...] = x_ref[...] + y * jax.nn.sigmoid(y)


def _tc_node(x, h, aggp, cntp, wh, wg, bn1, wn2, bn2):
    n, d = x.shape
    return pl.pallas_call(
        _node_body,
        out_shape=jax.ShapeDtypeStruct((n, d), F32),
    )(x, h, aggp, cntp, wh, wg, bn1, wn2, bn2)


_CH = 80
_NPAD = 10240


def _sc_gather_body(a_hbm, b_hbm, latc_hbm, src4_hbm, dst4_hbm, e2g4_hbm,
                    pre_hbm, idxs, idxd, idxg,
                    bufA0, bufB0, bufL0, bufA1, bufB1, bufL1,
                    semA0, semB0, semL0, semA1, semB1, semL1):
    cid = lax.axis_index("c")
    sid = lax.axis_index("s")
    wid = sid * 2 + cid
    chg = src4_hbm.shape[3]
    nchunks = src4_hbm.shape[1]
    k0 = wid * nchunks

    bufs = ((bufA0, bufB0, bufL0), (bufA1, bufB1, bufL1))
    sems = ((semA0, semB0, semL0), (semA1, semB1, semL1))

    pltpu.sync_copy(src4_hbm.at[wid], idxs)
    pltpu.sync_copy(dst4_hbm.at[wid], idxd)
    pltpu.sync_copy(e2g4_hbm.at[wid], idxg)

    def fire(k, b):
        bA, bB, bL = bufs[b]
        sA, sB, sL = sems[b]
        pltpu.async_copy(a_hbm.at[idxs.at[k, 0]], bA, sA)
        pltpu.async_copy(b_hbm.at[idxd.at[k, 0]], bB, sB)
        pltpu.async_copy(latc_hbm.at[idxg.at[k, 0]], bL, sL)

    def drain(b):
        bA, bB, bL = bufs[b]
        sA, sB, sL = sems[b]
        pltpu.make_async_copy(a_hbm.at[idxs.at[0, 0]], bA, sA).wait()
        pltpu.make_async_copy(b_hbm.at[idxd.at[0, 0]], bB, sB).wait()
        pltpu.make_async_copy(latc_hbm.at[idxg.at[0, 0]], bL, sL).wait()

    fire(0, 0)
    fire(1, 1)

    @pl.loop(0, nchunks + 1, step=2)
    def outer(g):
        for b in range(2):
            k = g + b

            @pl.when(k < nchunks)
            def _():
                drain(b)
                bA, bB, bL = bufs[b]

                @pl.loop(0, chg, unroll=4)
                def row(r):
                    for j in range(8):
                        sl = pl.ds(j * 16, 16)
                        bA[r, sl] = bA[r, sl] + bB[r, sl] + bL[r, sl]

                cb = (k0 + k) * chg
                pltpu.sync_copy(bA, pre_hbm.at[pl.ds(cb, chg)])

                @pl.when(k + 2 < nchunks)
                def _():
                    fire(k + 2, b)


def _sc_gather(A, B, latC, src4, dst4, e2g4):
    e = src4.shape[0] * src4.shape[1] * src4.shape[3]
    d = A.shape[1]
    nchunks = src4.shape[1]
    chg = src4.shape[3]
    mesh = plsc.VectorSubcoreMesh(core_axis_name="c", subcore_axis_name="s")
    f = pl.kernel(
        _sc_gather_body,
        out_type=jax.ShapeDtypeStruct((e, d), F32),
        mesh=mesh,
        scratch_types=[
            pltpu.VMEM((nchunks, 1, chg), jnp.int32),
            pltpu.VMEM((nchunks, 1, chg), jnp.int32),
            pltpu.VMEM((nchunks, 1, chg), jnp.int32),
            pltpu.VMEM((chg, d), F32),
            pltpu.VMEM((chg, d), F32),
            pltpu.VMEM((chg, d), F32),
            pltpu.VMEM((chg, d), F32),
            pltpu.VMEM((chg, d), F32),
            pltpu.VMEM((chg, d), F32),
            pltpu.SemaphoreType.DMA,
            pltpu.SemaphoreType.DMA,
            pltpu.SemaphoreType.DMA,
            pltpu.SemaphoreType.DMA,
            pltpu.SemaphoreType.DMA,
            pltpu.SemaphoreType.DMA,
        ],
    )
    return f(A, B, latC, src4, dst4, e2g4)


def _sc_count_body(src_hbm, zero_hbm, ones_hbm, cntp_hbm,
                   idxv, cbuf, onesv, shared_cnt):
    cid = lax.axis_index("c")
    sid = lax.axis_index("s")
    wid = sid * 2 + cid
    e = src_hbm.shape[0]
    per_w = e // 32
    nchunks = per_w // _CH
    rows_per_tile = _NPAD // 16

    pltpu.sync_copy(ones_hbm, onesv)
    pltpu.sync_copy(zero_hbm, cbuf)
    for j in range(rows_per_tile // _CH):
        zb = sid * rows_per_tile + j * _CH
        pltpu.sync_copy(cbuf, shared_cnt.at[pl.ds(zb, _CH)])
    plsc.subcore_barrier()

    def step(k, carry):
        cb = wid * per_w + k * _CH
        pltpu.sync_copy(src_hbm.at[pl.ds(cb, _CH)], idxv.at[0])
        pltpu.sync_copy(onesv, shared_cnt.at[idxv.at[0]], add=True)
        return carry

    lax.fori_loop(0, nchunks, step, 0)
    plsc.subcore_barrier()
    for j in range(rows_per_tile // _CH):
        zb = sid * rows_per_tile + j * _CH
        pltpu.sync_copy(shared_cnt.at[pl.ds(zb, _CH)], cbuf)
        pltpu.sync_copy(cbuf, cntp_hbm.at[cid, pl.ds(zb, _CH)])


def _sc_count(src, d):
    e = src.shape[0]
    mesh = plsc.VectorSubcoreMesh(core_axis_name="c", subcore_axis_name="s")
    zero = jnp.zeros((_CH, d), F32)
    ones = jnp.ones((_CH, d), F32)
    f = pl.kernel(
        _sc_count_body,
        out_type=jax.ShapeDtypeStruct((2, _NPAD, d), F32),
        mesh=mesh,
        scratch_types=[
            pltpu.VMEM((1, _CH), jnp.int32),
            pltpu.VMEM((_CH, d), F32),
            pltpu.VMEM((_CH, d), F32),
            pltpu.VMEM_SHARED((_NPAD, d), F32),
        ],
    )
    return f(src, zero, ones)


def _sc_scatter_body(ef2_hbm, src_hbm, zero_hbm, aggp_hbm,
                     idxv, vbuf, shared_agg):
    cid = lax.axis_index("c")
    sid = lax.axis_index("s")
    wid = sid * 2 + cid
    e = src_hbm.shape[0]
    per_w = e // 32
    nchunks = per_w // _CH
    rows_per_tile = _NPAD // 16

    pltpu.sync_copy(zero_hbm, vbuf)
    for j in range(rows_per_tile // _CH):
        zb = sid * rows_per_tile + j * _CH
        pltpu.sync_copy(vbuf, shared_agg.at[pl.ds(zb, _CH)])
    plsc.subcore_barrier()

    def step(k, carry):
        cb = wid * per_w + k * _CH
        pltpu.sync_copy(src_hbm.at[pl.ds(cb, _CH)], idxv.at[0])
        pltpu.sync_copy(ef2_hbm.at[pl.ds(cb, _CH)], vbuf)
        pltpu.sync_copy(vbuf, shared_agg.at[idxv.at[0]], add=True)
        return carry

    lax.fori_loop(0, nchunks, step, 0)
    plsc.subcore_barrier()

    for j in range(rows_per_tile // _CH):
        zb = sid * rows_per_tile + j * _CH
        pltpu.sync_copy(shared_agg.at[pl.ds(zb, _CH)], vbuf)
        pltpu.sync_copy(vbuf, aggp_hbm.at[cid, pl.ds(zb, _CH)])


def _sc_scatter(ef2, src):
    e, d = ef2.shape
    mesh = plsc.VectorSubcoreMesh(core_axis_name="c", subcore_axis_name="s")
    zero = jnp.zeros((_CH, d), F32)
    f = pl.kernel(
        _sc_scatter_body,
        out_type=jax.ShapeDtypeStruct((2, _NPAD, d), F32),
        mesh=mesh,
        scratch_types=[
            pltpu.VMEM((1, _CH), jnp.int32),
            pltpu.VMEM((_CH, d), F32),
            pltpu.VMEM_SHARED((_NPAD, d), F32),
        ],
    )
    return f(ef2, src, zero)


def kernel(node_features, lattices, edge_index, edge2graph, frac_diff,
           num_atoms, ln_g, ln_b, We1, be1, We2, be2, Wn1, bn1, Wn2, bn2):
    n, d = node_features.shape
    e = edge_index.shape[1]
    g = lattices.shape[0]
    ns = lattices.shape[1]
    diml = ns * ns

    Wa = We1[:d]
    Wb = We1[d:2 * d]
    Wl = We1[2 * d:2 * d + diml]
    Wf = jnp.zeros((8, d), F32).at[:ns].set(We1[2 * d + diml:])
    chg = 128
    epad = ((e + 32 * chg - 1) // (32 * chg)) * (32 * chg)
    fracp = jnp.zeros((epad, 8), F32).at[:e, :ns].set(frac_diff)
    ltl = (lattices @ jnp.swapaxes(lattices, -1, -2)).reshape(g, diml)
    src = edge_index[0]
    dst = edge_index[1]

    h, A, B, latC = _tc_prep(node_features, ln_g.reshape(1, d),
                             ln_b.reshape(1, d), ltl, Wa, Wb, Wl,
                             be1.reshape(1, d))

    nck = epad // chg // 32
    pz = jnp.zeros((epad - e,), jnp.int32)
    src4 = jnp.concatenate([src, pz]).reshape(32, nck, 1, chg)
    dst4 = jnp.concatenate([dst, pz]).reshape(32, nck, 1, chg)
    e2g4 = jnp.concatenate([edge2graph, pz]).reshape(32, nck, 1, chg)
    pre = _sc_gather(A, B, latC, src4, dst4, e2g4)
    cntp = _sc_count(src, d)

    ef2 = _tc_edge(pre, fracp, Wf, We2, be2.reshape(1, d))

    aggp = _sc_scatter(ef2, src)

    out = _tc_node(node_features, h, aggp, cntp, Wn1[:d], Wn1[d:],
                   bn1.reshape(1, d), Wn2, bn2.reshape(1, d))
    return out


# speedup vs baseline: 1.0975x; 1.0975x over previous
import functools
import jax
import jax.numpy as jnp
from jax import lax
from jax.experimental import pallas as pl
from jax.experimental.pallas import tpu as pltpu
from jax.experimental.pallas import tpu_sc as plsc

F32 = jnp.float32
BF16 = jnp.bfloat16


def _prep_body(x_ref, lng_ref, lnb_ref, ltl_ref, wa_ref, wb_ref, wl_ref,
               be1_ref, h_ref, a_ref, b_ref, latc_ref):
    x = x_ref[...]
    mu = jnp.mean(x, axis=-1, keepdims=True)
    var = jnp.mean((x - mu) ** 2, axis=-1, keepdims=True)
    h = (x - mu) * lax.rsqrt(var + 1e-5) * lng_ref[...] + lnb_ref[...]
    h_ref[...] = h
    a_ref[...] = jnp.dot(h, wa_ref[...], preferred_element_type=F32)
    b_ref[...] = jnp.dot(h, wb_ref[...], preferred_element_type=F32)
    latc_ref[...] = jnp.dot(ltl_ref[...], wl_ref[...],
                            preferred_element_type=F32) + be1_ref[...]


def _tc_prep(x, lng, lnb, ltl, wa, wb, wl, be1):
    n, d = x.shape
    g = ltl.shape[0]
    return pl.pallas_call(
        _prep_body,
        out_shape=(
            jax.ShapeDtypeStruct((n, d), F32),
            jax.ShapeDtypeStruct((n, d), F32),
            jax.ShapeDtypeStruct((n, d), F32),
            jax.ShapeDtypeStruct((g, d), F32),
        ),
    )(x, lng, lnb, ltl, wa, wb, wl, be1)


def _edge_body(pre_ref, fr_ref, wf_ref, w2_ref, be2_ref, out_ref):
    x = pre_ref[...] + jnp.dot(fr_ref[...], wf_ref[...],
                               preferred_element_type=F32)
    e1 = x * jax.nn.sigmoid(x)
    y = jnp.dot(e1, w2_ref[...], preferred_element_type=F32) + be2_ref[...]
    out_ref[...] = y * jax.nn.sigmoid(y)


def _tc_edge(pre, fracp, wf, w2, be2, blk=2048):
    e, d = pre.shape
    grid = e // blk
    return pl.pallas_call(
        _edge_body,
        grid=(grid,),
        in_specs=[
            pl.BlockSpec((blk, d), lambda i: (i, 0)),
            pl.BlockSpec((blk, 8), lambda i: (i, 0)),
            pl.BlockSpec((8, d), lambda i: (0, 0)),
            pl.BlockSpec((d, d), lambda i: (0, 0)),
            pl.BlockSpec((1, d), lambda i: (0, 0)),
        ],
        out_specs=pl.BlockSpec((blk, d), lambda i: (i, 0)),
        out_shape=jax.ShapeDtypeStruct((e, d), F32),
    )(pre, fracp, wf, w2, be2)


def _node_body(x_ref, h_ref, aggp_ref, cntp_ref, wh_ref, wg_ref, bn1_ref,
               wn2_ref, bn2_ref, out_ref):
    n = x_ref.shape[0]
    agg = aggp_ref[0, 0:n, :] + aggp_ref[1, 0:n, :]
    cnt = cntp_ref[0, 0:n, 0:1] + cntp_ref[1, 0:n, 0:1]
    agg = agg / jnp.maximum(cnt, 1.0)
    h = h_ref[...]
    t = (jnp.dot(h, wh_ref[...], preferred_element_type=F32)
         + jnp.dot(agg, wg_ref[...], preferred_element_type=F32)
         + bn1_ref[...])
    t = t * jax.nn.sigmoid(t)
    y = jnp.dot(t, wn2_ref[...], preferred_element_type=F32) + bn2_ref[...]
    out_ref[...] = x_ref[...] + y * jax.nn.sigmoid(y)


def _tc_node(x, h, aggp, cntp, wh, wg, bn1, wn2, bn2):
    n, d = x.shape
    return pl.pallas_call(
        _node_body,
        out_shape=jax.ShapeDtypeStruct((n, d), F32),
    )(x, h, aggp, cntp, wh, wg, bn1, wn2, bn2)


_CH = 80
_NPAD = 10240


def _sc_gather_body(a_hbm, b_hbm, latc_hbm, src4_hbm, dst4_hbm, e2g4_hbm,
                    pre_hbm, idxs, idxd, idxg,
                    bufA0, bufB0, bufL0, bufA1, bufB1, bufL1,
                    semA0, semB0, semL0, semA1, semB1, semL1):
    cid = lax.axis_index("c")
    sid = lax.axis_index("s")
    wid = sid * 2 + cid
    chg = src4_hbm.shape[3]
    nchunks = src4_hbm.shape[1]
    k0 = wid * nchunks

    bufs = ((bufA0, bufB0, bufL0), (bufA1, bufB1, bufL1))
    sems = ((semA0, semB0, semL0), (semA1, semB1, semL1))

    pltpu.sync_copy(src4_hbm.at[wid], idxs)
    pltpu.sync_copy(dst4_hbm.at[wid], idxd)
    pltpu.sync_copy(e2g4_hbm.at[wid], idxg)

    def fire(k, b):
        bA, bB, bL = bufs[b]
        sA, sB, sL = sems[b]
        pltpu.async_copy(a_hbm.at[idxs.at[k, 0]], bA, sA)
        pltpu.async_copy(b_hbm.at[idxd.at[k, 0]], bB, sB)
        pltpu.async_copy(latc_hbm.at[idxg.at[k, 0]], bL, sL)

    def drain(b):
        bA, bB, bL = bufs[b]
        sA, sB, sL = sems[b]
        pltpu.make_async_copy(a_hbm.at[idxs.at[0, 0]], bA, sA).wait()
        pltpu.make_async_copy(b_hbm.at[idxd.at[0, 0]], bB, sB).wait()
        pltpu.make_async_copy(latc_hbm.at[idxg.at[0, 0]], bL, sL).wait()

    fire(0, 0)
    fire(1, 1)

    @pl.loop(0, nchunks + 1, step=2)
    def outer(g):
        for b in range(2):
            k = g + b

            @pl.when(k < nchunks)
            def _():
                drain(b)
                bA, bB, bL = bufs[b]

                @pl.loop(0, chg, unroll=4)
                def row(r):
                    for j in range(8):
                        sl = pl.ds(j * 16, 16)
                        bA[r, sl] = bA[r, sl] + bB[r, sl] + bL[r, sl]

                cb = (k0 + k) * chg
                pltpu.sync_copy(bA, pre_hbm.at[pl.ds(cb, chg)])

                @pl.when(k + 2 < nchunks)
                def _():
                    fire(k + 2, b)


def _sc_gather(A, B, latC, src4, dst4, e2g4):
    e = src4.shape[0] * src4.shape[1] * src4.shape[3]
    d = A.shape[1]
    nchunks = src4.shape[1]
    chg = src4.shape[3]
    mesh = plsc.VectorSubcoreMesh(core_axis_name="c", subcore_axis_name="s")
    f = pl.kernel(
        _sc_gather_body,
        out_type=jax.ShapeDtypeStruct((e, d), F32),
        mesh=mesh,
        scratch_types=[
            pltpu.VMEM((nchunks, 1, chg), jnp.int32),
            pltpu.VMEM((nchunks, 1, chg), jnp.int32),
            pltpu.VMEM((nchunks, 1, chg), jnp.int32),
            pltpu.VMEM((chg, d), F32),
            pltpu.VMEM((chg, d), F32),
            pltpu.VMEM((chg, d), F32),
            pltpu.VMEM((chg, d), F32),
            pltpu.VMEM((chg, d), F32),
            pltpu.VMEM((chg, d), F32),
            pltpu.SemaphoreType.DMA,
            pltpu.SemaphoreType.DMA,
            pltpu.SemaphoreType.DMA,
            pltpu.SemaphoreType.DMA,
            pltpu.SemaphoreType.DMA,
            pltpu.SemaphoreType.DMA,
        ],
    )
    return f(A, B, latC, src4, dst4, e2g4)


def _sc_count_body(src_hbm, zero_hbm, ones_hbm, cntp_hbm,
                   idxv, cbuf, onesv, shared_cnt):
    cid = lax.axis_index("c")
    sid = lax.axis_index("s")
    wid = sid * 2 + cid
    e = src_hbm.shape[0]
    per_w = e // 32
    nchunks = per_w // _CH
    rows_per_tile = _NPAD // 16

    pltpu.sync_copy(ones_hbm, onesv)
    pltpu.sync_copy(zero_hbm, cbuf)
    for j in range(rows_per_tile // _CH):
        zb = sid * rows_per_tile + j * _CH
        pltpu.sync_copy(cbuf, shared_cnt.at[pl.ds(zb, _CH)])
    plsc.subcore_barrier()

    def step(k, carry):
        cb = wid * per_w + k * _CH
        pltpu.sync_copy(src_hbm.at[pl.ds(cb, _CH)], idxv.at[0])
        pltpu.sync_copy(onesv, shared_cnt.at[idxv.at[0]], add=True)
        return carry

    lax.fori_loop(0, nchunks, step, 0)
    plsc.subcore_barrier()
    for j in range(rows_per_tile // _CH):
        zb = sid * rows_per_tile + j * _CH
        pltpu.sync_copy(shared_cnt.at[pl.ds(zb, _CH)], cbuf)
        pltpu.sync_copy(cbuf, cntp_hbm.at[cid, pl.ds(zb, _CH)])


def _sc_count(src, d):
    e = src.shape[0]
    mesh = plsc.VectorSubcoreMesh(core_axis_name="c", subcore_axis_name="s")
    zero = jnp.zeros((_CH, d), F32)
    ones = jnp.ones((_CH, d), F32)
    f = pl.kernel(
        _sc_count_body,
        out_type=jax.ShapeDtypeStruct((2, _NPAD, d), F32),
        mesh=mesh,
        scratch_types=[
            pltpu.VMEM((1, _CH), jnp.int32),
            pltpu.VMEM((_CH, d), F32),
            pltpu.VMEM((_CH, d), F32),
            pltpu.VMEM_SHARED((_NPAD, d), F32),
        ],
    )
    return f(src, zero, ones)


def _sc_scatter_body(ef2_hbm, src4_hbm, zero_hbm, aggp_hbm,
                     idxa, vbuf0, vbuf1, shared_agg, sem0, sem1):
    cid = lax.axis_index("c")
    sid = lax.axis_index("s")
    wid = sid * 2 + cid
    nchunks = src4_hbm.shape[1]
    k0 = wid * nchunks
    rows_per_tile = _NPAD // 16
    vbufs = (vbuf0, vbuf1)
    sems = (sem0, sem1)

    pltpu.sync_copy(src4_hbm.at[wid], idxa)
    pltpu.sync_copy(zero_hbm, vbuf0)
    for j in range(rows_per_tile // _CH):
        zb = sid * rows_per_tile + j * _CH
        pltpu.sync_copy(vbuf0, shared_agg.at[pl.ds(zb, _CH)])
    plsc.subcore_barrier()

    def fire(k, b):
        cb = (k0 + k) * _CH
        pltpu.async_copy(ef2_hbm.at[pl.ds(cb, _CH)], vbufs[b], sems[b])

    def drain(b):
        pltpu.make_async_copy(
            ef2_hbm.at[pl.ds(0, _CH)], vbufs[b], sems[b]).wait()

    fire(0, 0)
    fire(1, 1)

    @pl.loop(0, nchunks + 1, step=2)
    def outer(g):
        for b in range(2):
            k = g + b

            @pl.when(k < nchunks)
            def _():
                drain(b)
                pltpu.sync_copy(vbufs[b], shared_agg.at[idxa.at[k, 0]],
                                add=True)

                @pl.when(k + 2 < nchunks)
                def _():
                    fire(k + 2, b)

    plsc.subcore_barrier()

    for j in range(rows_per_tile // _CH):
        zb = sid * rows_per_tile + j * _CH
        pltpu.sync_copy(shared_agg.at[pl.ds(zb, _CH)], vbuf0)
        pltpu.sync_copy(vbuf0, aggp_hbm.at[cid, pl.ds(zb, _CH)])


def _sc_scatter(ef2, src4s):
    d = ef2.shape[1]
    nchunks = src4s.shape[1]
    mesh = plsc.VectorSubcoreMesh(core_axis_name="c", subcore_axis_name="s")
    zero = jnp.zeros((_CH, d), F32)
    f = pl.kernel(
        _sc_scatter_body,
        out_type=jax.ShapeDtypeStruct((2, _NPAD, d), F32),
        mesh=mesh,
        scratch_types=[
            pltpu.VMEM((nchunks, 1, _CH), jnp.int32),
            pltpu.VMEM((_CH, d), F32),
            pltpu.VMEM((_CH, d), F32),
            pltpu.VMEM_SHARED((_NPAD, d), F32),
            pltpu.SemaphoreType.DMA,
            pltpu.SemaphoreType.DMA,
        ],
    )
    return f(ef2, src4s, zero)


def kernel(node_features, lattices, edge_index, edge2graph, frac_diff,
           num_atoms, ln_g, ln_b, We1, be1, We2, be2, Wn1, bn1, Wn2, bn2):
    n, d = node_features.shape
    e = edge_index.shape[1]
    g = lattices.shape[0]
    ns = lattices.shape[1]
    diml = ns * ns

    Wa = We1[:d]
    Wb = We1[d:2 * d]
    Wl = We1[2 * d:2 * d + diml]
    Wf = jnp.zeros((8, d), F32).at[:ns].set(We1[2 * d + diml:])
    chg = 128
    epad = ((e + 32 * chg - 1) // (32 * chg)) * (32 * chg)
    fracp = jnp.zeros((epad, 8), F32).at[:e, :ns].set(frac_diff)
    ltl = (lattices @ jnp.swapaxes(lattices, -1, -2)).reshape(g, diml)
    src = edge_index[0]
    dst = edge_index[1]

    h, A, B, latC = _tc_prep(node_features, ln_g.reshape(1, d),
                             ln_b.reshape(1, d), ltl, Wa, Wb, Wl,
                             be1.reshape(1, d))

    nck = epad // chg // 32
    pz = jnp.zeros((epad - e,), jnp.int32)
    src4 = jnp.concatenate([src, pz]).reshape(32, nck, 1, chg)
    dst4 = jnp.concatenate([dst, pz]).reshape(32, nck, 1, chg)
    e2g4 = jnp.concatenate([edge2graph, pz]).reshape(32, nck, 1, chg)
    pre = _sc_gather(A, B, latC, src4, dst4, e2g4)
    cntp = _sc_count(src, d)

    ef2 = _tc_edge(pre, fracp, Wf, We2, be2.reshape(1, d))

    src4s = src.reshape(32, e // _CH // 32, 1, _CH)
    aggp = _sc_scatter(ef2, src4s)

    out = _tc_node(node_features, h, aggp, cntp, Wn1[:d], Wn1[d:],
                   bn1.reshape(1, d), Wn2, bn2.reshape(1, d))
    return out


# speedup vs baseline: 1.1101x; 1.0114x over previous
import functools
import jax
import jax.numpy as jnp
from jax import lax
from jax.experimental import pallas as pl
from jax.experimental.pallas import tpu as pltpu
from jax.experimental.pallas import tpu_sc as plsc

F32 = jnp.float32
BF16 = jnp.bfloat16


def _prep_body(x_ref, lng_ref, lnb_ref, ltl_ref, wa_ref, wb_ref, wl_ref,
               be1_ref, h_ref, a_ref, b_ref, latc_ref):
    x = x_ref[...]
    mu = jnp.mean(x, axis=-1, keepdims=True)
    var = jnp.mean((x - mu) ** 2, axis=-1, keepdims=True)
    h = (x - mu) * lax.rsqrt(var + 1e-5) * lng_ref[...] + lnb_ref[...]
    h_ref[...] = h
    a_ref[...] = jnp.dot(h, wa_ref[...], preferred_element_type=F32)
    b_ref[...] = jnp.dot(h, wb_ref[...], preferred_element_type=F32)
    latc_ref[...] = jnp.dot(ltl_ref[...], wl_ref[...],
                            preferred_element_type=F32) + be1_ref[...]


def _tc_prep(x, lng, lnb, ltl, wa, wb, wl, be1):
    n, d = x.shape
    g = ltl.shape[0]
    return pl.pallas_call(
        _prep_body,
        out_shape=(
            jax.ShapeDtypeStruct((n, d), F32),
            jax.ShapeDtypeStruct((n, d), F32),
            jax.ShapeDtypeStruct((n, d), F32),
            jax.ShapeDtypeStruct((g, d), F32),
        ),
    )(x, lng, lnb, ltl, wa, wb, wl, be1)


def _edge_body(pre_ref, fr_ref, wf_ref, w2_ref, be2_ref, out_ref):
    x = pre_ref[...] + jnp.dot(fr_ref[...], wf_ref[...],
                               preferred_element_type=F32)
    e1 = x * jax.nn.sigmoid(x)
    y = jnp.dot(e1, w2_ref[...], preferred_element_type=F32) + be2_ref[...]
    out_ref[...] = y * jax.nn.sigmoid(y)


def _tc_edge(pre, fracp, wf, w2, be2, blk=2048):
    e, d = pre.shape
    grid = e // blk
    return pl.pallas_call(
        _edge_body,
        grid=(grid,),
        in_specs=[
            pl.BlockSpec((blk, d), lambda i: (i, 0)),
            pl.BlockSpec((blk, 8), lambda i: (i, 0)),
            pl.BlockSpec((8, d), lambda i: (0, 0)),
            pl.BlockSpec((d, d), lambda i: (0, 0)),
            pl.BlockSpec((1, d), lambda i: (0, 0)),
        ],
        out_specs=pl.BlockSpec((blk, d), lambda i: (i, 0)),
        out_shape=jax.ShapeDtypeStruct((e, d), F32),
    )(pre, fracp, wf, w2, be2)


def _node_body(x_ref, h_ref, aggp_ref, cntp_ref, wh_ref, wg_ref, bn1_ref,
               wn2_ref, bn2_ref, out_ref):
    n = x_ref.shape[0]
    agg = aggp_ref[0, 0:n, :] + aggp_ref[1, 0:n, :]
    cnt = cntp_ref[0, 0:n, 0:1] + cntp_ref[1, 0:n, 0:1]
    agg = agg / jnp.maximum(cnt, 1.0)
    h = h_ref[...]
    t = (jnp.dot(h, wh_ref[...], preferred_element_type=F32)
         + jnp.dot(agg, wg_ref[...], preferred_element_type=F32)
         + bn1_ref[...])
    t = t * jax.nn.sigmoid(t)
    y = jnp.dot(t, wn2_ref[...], preferred_element_type=F32) + bn2_ref[...]
    out_ref[...] = x_ref[...] + y * jax.nn.sigmoid(y)


def _tc_node(x, h, aggp, cntp, wh, wg, bn1, wn2, bn2):
    n, d = x.shape
    return pl.pallas_call(
        _node_body,
        out_shape=jax.ShapeDtypeStruct((n, d), F32),
    )(x, h, aggp, cntp, wh, wg, bn1, wn2, bn2)


_CH = 80
_NPAD = 10240


def _sc_gather_body(a_hbm, b_hbm, latc_hbm, src4_hbm, dst4_hbm, e2g4_hbm,
                    pre_hbm, idxs, idxd, idxg,
                    bufA0, bufB0, bufL0, bufA1, bufB1, bufL1,
                    semA0, semB0, semL0, semA1, semB1, semL1):
    cid = lax.axis_index("c")
    sid = lax.axis_index("s")
    wid = sid * 2 + cid
    chg = src4_hbm.shape[3]
    nchunks = src4_hbm.shape[1]
    k0 = wid * nchunks

    bufs = ((bufA0, bufB0, bufL0), (bufA1, bufB1, bufL1))
    sems = ((semA0, semB0, semL0), (semA1, semB1, semL1))

    pltpu.sync_copy(src4_hbm.at[wid], idxs)
    pltpu.sync_copy(dst4_hbm.at[wid], idxd)
    pltpu.sync_copy(e2g4_hbm.at[wid], idxg)

    def fire(k, b):
        bA, bB, bL = bufs[b]
        sA, sB, sL = sems[b]
        pltpu.async_copy(a_hbm.at[idxs.at[k, 0]], bA, sA)
        pltpu.async_copy(b_hbm.at[idxd.at[k, 0]], bB, sB)
        pltpu.async_copy(latc_hbm.at[idxg.at[k, 0]], bL, sL)

    def drain(b):
        bA, bB, bL = bufs[b]
        sA, sB, sL = sems[b]
        pltpu.make_async_copy(a_hbm.at[idxs.at[0, 0]], bA, sA).wait()
        pltpu.make_async_copy(b_hbm.at[idxd.at[0, 0]], bB, sB).wait()
        pltpu.make_async_copy(latc_hbm.at[idxg.at[0, 0]], bL, sL).wait()

    fire(0, 0)
    fire(1, 1)

    @pl.loop(0, nchunks + 1, step=2)
    def outer(g):
        for b in range(2):
            k = g + b

            @pl.when(k < nchunks)
            def _():
                drain(b)
                bA, bB, bL = bufs[b]

                @pl.loop(0, chg, unroll=4)
                def row(r):
                    for j in range(8):
                        sl = pl.ds(j * 16, 16)
                        bA[r, sl] = bA[r, sl] + bB[r, sl] + bL[r, sl]

                cb = (k0 + k) * chg
                pltpu.sync_copy(bA, pre_hbm.at[pl.ds(cb, chg)])

                @pl.when(k + 2 < nchunks)
                def _():
                    fire(k + 2, b)


def _sc_gather(A, B, latC, src4, dst4, e2g4):
    e = src4.shape[0] * src4.shape[1] * src4.shape[3]
    d = A.shape[1]
    nchunks = src4.shape[1]
    chg = src4.shape[3]
    mesh = plsc.VectorSubcoreMesh(core_axis_name="c", subcore_axis_name="s")
    f = pl.kernel(
        _sc_gather_body,
        out_type=jax.ShapeDtypeStruct((e, d), F32),
        mesh=mesh,
        scratch_types=[
            pltpu.VMEM((nchunks, 1, chg), jnp.int32),
            pltpu.VMEM((nchunks, 1, chg), jnp.int32),
            pltpu.VMEM((nchunks, 1, chg), jnp.int32),
            pltpu.VMEM((chg, d), F32),
            pltpu.VMEM((chg, d), F32),
            pltpu.VMEM((chg, d), F32),
            pltpu.VMEM((chg, d), F32),
            pltpu.VMEM((chg, d), F32),
            pltpu.VMEM((chg, d), F32),
            pltpu.SemaphoreType.DMA,
            pltpu.SemaphoreType.DMA,
            pltpu.SemaphoreType.DMA,
            pltpu.SemaphoreType.DMA,
            pltpu.SemaphoreType.DMA,
            pltpu.SemaphoreType.DMA,
        ],
    )
    return f(A, B, latC, src4, dst4, e2g4)


def _sc_count_body(src4_hbm, zero_hbm, ones_hbm, cntp_hbm,
                   idxa, cbuf, onesv, shared_cnt):
    cid = lax.axis_index("c")
    sid = lax.axis_index("s")
    wid = sid * 2 + cid
    nchunks = src4_hbm.shape[1]
    rows_per_tile = _NPAD // 16

    pltpu.sync_copy(src4_hbm.at[wid], idxa)
    pltpu.sync_copy(ones_hbm, onesv)
    pltpu.sync_copy(zero_hbm, cbuf)
    for j in range(rows_per_tile // _CH):
        zb = sid * rows_per_tile + j * _CH
        pltpu.sync_copy(cbuf, shared_cnt.at[pl.ds(zb, _CH)])
    plsc.subcore_barrier()

    @pl.loop(0, nchunks)
    def step(k):
        pltpu.sync_copy(onesv, shared_cnt.at[idxa.at[k, 0]], add=True)

    plsc.subcore_barrier()
    for j in range(rows_per_tile // _CH):
        zb = sid * rows_per_tile + j * _CH
        pltpu.sync_copy(shared_cnt.at[pl.ds(zb, _CH)], cbuf)
        pltpu.sync_copy(cbuf, cntp_hbm.at[cid, pl.ds(zb, _CH)])


def _sc_count(src4s, d):
    nchunks = src4s.shape[1]
    mesh = plsc.VectorSubcoreMesh(core_axis_name="c", subcore_axis_name="s")
    zero = jnp.zeros((_CH, d), F32)
    ones = jnp.ones((_CH, d), F32)
    f = pl.kernel(
        _sc_count_body,
        out_type=jax.ShapeDtypeStruct((2, _NPAD, d), F32),
        mesh=mesh,
        scratch_types=[
            pltpu.VMEM((nchunks, 1, _CH), jnp.int32),
            pltpu.VMEM((_CH, d), F32),
            pltpu.VMEM((_CH, d), F32),
            pltpu.VMEM_SHARED((_NPAD, d), F32),
        ],
    )
    return f(src4s, zero, ones)


def _sc_scatter_body(ef2_hbm, src4_hbm, zero_hbm, aggp_hbm,
                     idxa, vbuf0, vbuf1, shared_agg, sem0, sem1):
    cid = lax.axis_index("c")
    sid = lax.axis_index("s")
    wid = sid * 2 + cid
    nchunks = src4_hbm.shape[1]
    k0 = wid * nchunks
    rows_per_tile = _NPAD // 16
    vbufs = (vbuf0, vbuf1)
    sems = (sem0, sem1)

    pltpu.sync_copy(src4_hbm.at[wid], idxa)
    pltpu.sync_copy(zero_hbm, vbuf0)
    for j in range(rows_per_tile // _CH):
        zb = sid * rows_per_tile + j * _CH
        pltpu.sync_copy(vbuf0, shared_agg.at[pl.ds(zb, _CH)])
    plsc.subcore_barrier()

    def fire(k, b):
        cb = (k0 + k) * _CH
        pltpu.async_copy(ef2_hbm.at[pl.ds(cb, _CH)], vbufs[b], sems[b])

    def drain(b):
        pltpu.make_async_copy(
            ef2_hbm.at[pl.ds(0, _CH)], vbufs[b], sems[b]).wait()

    fire(0, 0)
    fire(1, 1)

    @pl.loop(0, nchunks + 1, step=2)
    def outer(g):
        for b in range(2):
            k = g + b

            @pl.when(k < nchunks)
            def _():
                drain(b)
                pltpu.sync_copy(vbufs[b], shared_agg.at[idxa.at[k, 0]],
                                add=True)

                @pl.when(k + 2 < nchunks)
                def _():
                    fire(k + 2, b)

    plsc.subcore_barrier()

    for j in range(rows_per_tile // _CH):
        zb = sid * rows_per_tile + j * _CH
        pltpu.sync_copy(shared_agg.at[pl.ds(zb, _CH)], vbuf0)
        pltpu.sync_copy(vbuf0, aggp_hbm.at[cid, pl.ds(zb, _CH)])


def _sc_scatter(ef2, src4s):
    d = ef2.shape[1]
    nchunks = src4s.shape[1]
    mesh = plsc.VectorSubcoreMesh(core_axis_name="c", subcore_axis_name="s")
    zero = jnp.zeros((_CH, d), F32)
    f = pl.kernel(
        _sc_scatter_body,
        out_type=jax.ShapeDtypeStruct((2, _NPAD, d), F32),
        mesh=mesh,
        scratch_types=[
            pltpu.VMEM((nchunks, 1, _CH), jnp.int32),
            pltpu.VMEM((_CH, d), F32),
            pltpu.VMEM((_CH, d), F32),
            pltpu.VMEM_SHARED((_NPAD, d), F32),
            pltpu.SemaphoreType.DMA,
            pltpu.SemaphoreType.DMA,
        ],
    )
    return f(ef2, src4s, zero)


def kernel(node_features, lattices, edge_index, edge2graph, frac_diff,
           num_atoms, ln_g, ln_b, We1, be1, We2, be2, Wn1, bn1, Wn2, bn2):
    n, d = node_features.shape
    e = edge_index.shape[1]
    g = lattices.shape[0]
    ns = lattices.shape[1]
    diml = ns * ns

    Wa = We1[:d]
    Wb = We1[d:2 * d]
    Wl = We1[2 * d:2 * d + diml]
    Wf = jnp.zeros((8, d), F32).at[:ns].set(We1[2 * d + diml:])
    chg = 128
    epad = ((e + 32 * chg - 1) // (32 * chg)) * (32 * chg)
    fracp = jnp.zeros((epad, 8), F32).at[:e, :ns].set(frac_diff)
    ltl = (lattices @ jnp.swapaxes(lattices, -1, -2)).reshape(g, diml)
    src = edge_index[0]
    dst = edge_index[1]

    h, A, B, latC = _tc_prep(node_features, ln_g.reshape(1, d),
                             ln_b.reshape(1, d), ltl, Wa, Wb, Wl,
                             be1.reshape(1, d))

    nck = epad // chg // 32
    pz = jnp.zeros((epad - e,), jnp.int32)
    src4 = jnp.concatenate([src, pz]).reshape(32, nck, 1, chg)
    dst4 = jnp.concatenate([dst, pz]).reshape(32, nck, 1, chg)
    e2g4 = jnp.concatenate([edge2graph, pz]).reshape(32, nck, 1, chg)
    pre = _sc_gather(A, B, latC, src4, dst4, e2g4)
    src4s = src.reshape(32, e // _CH // 32, 1, _CH)
    cntp = _sc_count(src4s, d)

    ef2 = _tc_edge(pre, fracp, Wf, We2, be2.reshape(1, d))

    aggp = _sc_scatter(ef2, src4s)

    out = _tc_node(node_features, h, aggp, cntp, Wn1[:d], Wn1[d:],
                   bn1.reshape(1, d), Wn2, bn2.reshape(1, d))
    return out
